# baseline (speedup 1.0000x reference)
import functools

import numpy as np
import jax
import jax.numpy as jnp
from jax import lax
from jax.experimental import pallas as pl
from jax.experimental.pallas import tpu as pltpu
from jax.experimental.pallas import tpu_sc as plsc

_V = 1000000
_D = 128
_ND = 13
_NS = 26
_NF = _NS + 1
_GRP = 8
_GR = _GRP * _NF

_row_s = np.zeros((_GR,), dtype=np.int64)
_row_f = np.zeros((_GR,), dtype=np.int64)
for _i in range(_GR):
    if _i < _GRP:
        _row_s[_i], _row_f[_i] = _i, 0
    else:
        _row_s[_i], _row_f[_i] = (_i - _GRP) % _GRP, 1 + (_i - _GRP) // _GRP
_MASK = (_row_s[:, None] == _row_s[None, :]).astype(np.float32)
_SREP = (_row_f[:, None] == np.arange(32)[None, :]).astype(np.float32)

_NQ = 7
_EQ = np.zeros((4, 32, _D), dtype=np.float32)
for _q in range(4):
    for _m in range(32):
        _EQ[_q, _m, 32 * _q + _m] = 1.0

_iu0, _iu1 = np.triu_indices(_NF)
_SELP = np.zeros((378, _NQ, _D), dtype=np.float32)
for _p in range(378):
    _f, _m = _iu0[_p], _iu1[_p]
    _SELP[_p, _f // 4, 32 * (_f % 4) + _m] = 1.0


def _sc_gather(table, idx):
    rows = idx.shape[0]
    info = plsc.get_sparse_core_info()
    nw = info.num_cores * info.num_subcores
    per_w = rows // nw
    ch = 256 if per_w % 256 == 0 else 208
    nch = per_w // ch
    mesh = plsc.VectorSubcoreMesh(core_axis_name="c", subcore_axis_name="s")

    @functools.partial(
        pl.kernel,
        out_type=jax.ShapeDtypeStruct((rows, _D), jnp.float32),
        mesh=mesh,
        scratch_types=[
            pltpu.VMEM((per_w,), jnp.int32),
            pltpu.VMEM((2, ch, _D), jnp.float32),
            pltpu.SemaphoreType.DMA,
            pltpu.SemaphoreType.DMA,
        ],
    )
    def gather_kernel(table_hbm, idx_hbm, out_hbm, idx_v, buf, gsem, osem):
        wid = lax.axis_index("s") * info.num_cores + lax.axis_index("c")
        base = wid * per_w
        pltpu.sync_copy(idx_hbm.at[pl.ds(base, per_w)], idx_v)
        gathers = [None] * nch
        outs = [None] * nch
        gathers[0] = pltpu.async_copy(
            table_hbm.at[idx_v.at[pl.ds(0, ch)]], buf.at[0], gsem
        )
        for c in range(nch):
            if c + 1 < nch:
                if c >= 1:
                    outs[c - 1].wait()
                gathers[c + 1] = pltpu.async_copy(
                    table_hbm.at[idx_v.at[pl.ds((c + 1) * ch, ch)]],
                    buf.at[(c + 1) % 2],
                    gsem,
                )
            gathers[c].wait()
            outs[c] = pltpu.async_copy(
                buf.at[c % 2], out_hbm.at[pl.ds(base + c * ch, ch)], osem
            )
        outs[nch - 1].wait()
        if nch >= 2:
            outs[nch - 2].wait()

    return gather_kernel(table, idx)


def _mm_t(a, w):
    return lax.dot_general(
        a, w, (((1,), (1,)), ((), ())), preferred_element_type=jnp.float32
    )


_BF = jnp.bfloat16


def _dense_body(
    xd_ref, emb_ref,
    bw0, bb0, bw1, bb1, bw2, bb2,
    mask_ref, srep_ref, eq_ref,
    w0h, wsym, tb0, tw1, tb1, tw2, tb2, tw3, tb3, tw4, tb4,
    out_ref,
    h_ref, g_ref,
):
    bb = xd_ref.shape[0]
    ngrp = bb // _GRP
    h = xd_ref[...]
    h = jnp.maximum(_mm_t(h, bw0[...]) + bb0[...], 0.0)
    h = jnp.maximum(_mm_t(h, bw1[...]) + bb1[...], 0.0)
    h = jnp.maximum(_mm_t(h, bw2[...]) + bb2[...], 0.0)
    h_bf = h.astype(_BF)
    h_ref[...] = h
    mask = mask_ref[...]
    srep = srep_ref[...]

    def grp(g, _):
        hg = h_ref[pl.ds(g * _GRP, _GRP), :]
        eg = jnp.reshape(
            emb_ref[:, pl.ds(g * _GRP, _GRP), :], (_NS * _GRP, _D)
        )
        rg = jnp.concatenate([hg, eg], axis=0).astype(_BF)
        mg = _mm_t(rg, rg).astype(_BF) * mask
        gc = lax.dot_general(
            mg, srep, (((1,), (0,)), ((), ())),
            preferred_element_type=jnp.float32,
        )
        g_ref[0, pl.ds(g * _GRP, _GRP), :] = gc[: _GRP, :]
        g_ref[1:, pl.ds(g * _GRP, _GRP), :] = jnp.reshape(
            gc[_GRP:, :], (_NS, _GRP, 32)
        )
        return 0

    lax.fori_loop(0, ngrp, grp, 0, unroll=32)

    z = _mm_t(h_bf, w0h[...])
    for a in range(_NQ):
        pa = None
        for q in range(4):
            f = 4 * a + q
            if f >= _NF:
                break
            t = lax.dot_general(
                g_ref[f].astype(_BF), eq_ref[q],
                (((1,), (0,)), ((), ())),
                preferred_element_type=jnp.float32,
            )
            pa = t if pa is None else pa + t
        z = z + _mm_t(pa.astype(_BF), wsym[pl.ds(a * 1024, 1024), :])
    z = jnp.maximum(z + tb0[...], 0.0)
    z = jnp.maximum(_mm_t(z.astype(_BF), tw1[...]) + tb1[...], 0.0)
    z = jnp.maximum(_mm_t(z.astype(_BF), tw2[...]) + tb2[...], 0.0)
    z = jnp.maximum(_mm_t(z.astype(_BF), tw3[...]) + tb3[...], 0.0)
    out_ref[...] = _mm_t(z.astype(_BF), tw4[...]) + tb4[0, 0]


def _tc_dense(xd, emb3, bw0, bb0, bw1, bb1, bw2, bb2, mask, srep, eq,
              w0h, wsym, tb0, tw1, tb1, tw2, tb2, tw3, tb3, tw4, tb4):
    b = xd.shape[0]
    bb = 512
    nblk = b // bb

    def _full(a):
        return pl.BlockSpec(a.shape, lambda i, _n=None, _nd=a.ndim: (0,) * _nd)

    ws = (bw0, bb0, bw1, bb1, bw2, bb2, mask, srep, eq,
          w0h, wsym, tb0, tw1, tb1, tw2, tb2, tw3, tb3, tw4, tb4)
    return pl.pallas_call(
        _dense_body,
        grid=(nblk,),
        in_specs=[
            pl.BlockSpec((bb, _ND), lambda i: (i, 0)),
            pl.BlockSpec((_NS, bb, _D), lambda i: (0, i, 0)),
        ] + [_full(w) for w in ws],
        out_specs=pl.BlockSpec((bb, 8), lambda i: (i, 0)),
        out_shape=jax.ShapeDtypeStruct((b, 8), jnp.float32),
        scratch_shapes=[
            pltpu.VMEM((bb, _D), jnp.float32),
            pltpu.VMEM((_NF, bb, 32), jnp.float32),
        ],
    )(xd, emb3, *ws)


def kernel(x, table, bot_w0, bot_b0, bot_w1, bot_b1, bot_w2, bot_b2,
           top_w0, top_b0, top_w1, top_b1, top_w2, top_b2,
           top_w3, top_b3, top_w4, top_b4):
    b = x.shape[0]
    xd = x[:, :_ND]
    idxt = x[:, _ND:].astype(jnp.int32).T % _V
    bf = lambda v: v.astype(_BF)
    w0h = bf(top_w0[:, :_D])
    wsym = jnp.einsum(
        "op,pam->aom", bf(top_w0[:, _D:]), jnp.asarray(_SELP, _BF),
        preferred_element_type=_BF,
    )
    wsym = jnp.reshape(wsym, (_NQ * 1024, _D))
    tw4p = jnp.zeros((8, top_w4.shape[1]), jnp.float32).at[0].set(top_w4[0])
    r2 = lambda v: jnp.reshape(v, (1, -1))
    wargs = (
        bot_w0, r2(bot_b0), bot_w1, r2(bot_b1), bot_w2, r2(bot_b2),
        bf(jnp.asarray(_MASK)), bf(jnp.asarray(_SREP)), bf(jnp.asarray(_EQ)),
        w0h, wsym, r2(top_b0), bf(top_w1), r2(top_b1), bf(top_w2), r2(top_b2),
        bf(top_w3), r2(top_b3), bf(tw4p), r2(top_b4),
    )
    hb = b // 2
    outs = []
    for i in range(2):
        idx_i = jnp.reshape(idxt[:, i * hb : (i + 1) * hb], (-1,))
        emb_i = _sc_gather(table, idx_i)
        emb3_i = jnp.reshape(emb_i, (_NS, hb, _D))
        outs.append(_tc_dense(xd[i * hb : (i + 1) * hb], emb3_i, *wargs))
    return jnp.concatenate(outs, axis=0)[:, :1]


# speedup vs baseline: 1.0223x; 1.0223x over previous
import functools

import numpy as np
import jax
import jax.numpy as jnp
from jax import lax
from jax.experimental import pallas as pl
from jax.experimental.pallas import tpu as pltpu
from jax.experimental.pallas import tpu_sc as plsc

_V = 1000000
_D = 128
_ND = 13
_NS = 26
_NF = _NS + 1
_GRP = 8
_GR = _GRP * _NF

_row_s = np.zeros((_GR,), dtype=np.int64)
_row_f = np.zeros((_GR,), dtype=np.int64)
for _i in range(_GR):
    if _i < _GRP:
        _row_s[_i], _row_f[_i] = _i, 0
    else:
        _row_s[_i], _row_f[_i] = (_i - _GRP) % _GRP, 1 + (_i - _GRP) // _GRP
_MASK = (_row_s[:, None] == _row_s[None, :]).astype(np.float32)
_SREP = (_row_f[:, None] == np.arange(32)[None, :]).astype(np.float32)

_NQ = 7
_EQ = np.zeros((4, 32, _D), dtype=np.float32)
for _q in range(4):
    for _m in range(32):
        _EQ[_q, _m, 32 * _q + _m] = 1.0

_iu0, _iu1 = np.triu_indices(_NF)
_SELP = np.zeros((378, _NQ, _D), dtype=np.float32)
for _p in range(378):
    _f, _m = _iu0[_p], _iu1[_p]
    _SELP[_p, _f // 4, 32 * (_f % 4) + _m] = 1.0


def _sc_gather(table, idx):
    rows = idx.shape[0]
    info = plsc.get_sparse_core_info()
    nw = info.num_cores * info.num_subcores
    per_w = rows // nw
    ch = 256 if per_w % 256 == 0 else 208
    nch = per_w // ch
    mesh = plsc.VectorSubcoreMesh(core_axis_name="c", subcore_axis_name="s")

    @functools.partial(
        pl.kernel,
        out_type=jax.ShapeDtypeStruct((rows, _D), jnp.float32),
        mesh=mesh,
        scratch_types=[
            pltpu.VMEM((per_w,), jnp.int32),
            pltpu.VMEM((2, ch, _D), jnp.float32),
            pltpu.SemaphoreType.DMA,
            pltpu.SemaphoreType.DMA,
        ],
    )
    def gather_kernel(table_hbm, idx_hbm, out_hbm, idx_v, buf, gsem, osem):
        wid = lax.axis_index("s") * info.num_cores + lax.axis_index("c")
        base = wid * per_w
        pltpu.sync_copy(idx_hbm.at[pl.ds(base, per_w)], idx_v)
        gathers = [None] * nch
        outs = [None] * nch
        gathers[0] = pltpu.async_copy(
            table_hbm.at[idx_v.at[pl.ds(0, ch)]], buf.at[0], gsem
        )
        for c in range(nch):
            if c + 1 < nch:
                if c >= 1:
                    outs[c - 1].wait()
                gathers[c + 1] = pltpu.async_copy(
                    table_hbm.at[idx_v.at[pl.ds((c + 1) * ch, ch)]],
                    buf.at[(c + 1) % 2],
                    gsem,
                )
            gathers[c].wait()
            outs[c] = pltpu.async_copy(
                buf.at[c % 2], out_hbm.at[pl.ds(base + c * ch, ch)], osem
            )
        outs[nch - 1].wait()
        if nch >= 2:
            outs[nch - 2].wait()

    return gather_kernel(table, idx)


def _mm_t(a, w):
    return lax.dot_general(
        a, w, (((1,), (1,)), ((), ())), preferred_element_type=jnp.float32
    )


_BF = jnp.bfloat16


def _dense_body(
    xd_ref, emb_ref,
    bw0, bb0, bw1, bb1, bw2, bb2,
    mask_ref, srep_ref, eq_ref,
    w0h, wsym, tb0, tw1, tb1, tw2, tb2, tw3, tb3, tw4, tb4,
    out_ref,
    h_ref, g_ref,
):
    bb = xd_ref.shape[0]
    ngrp = bb // _GRP
    h = xd_ref[...]
    h = jnp.maximum(_mm_t(h, bw0[...]) + bb0[...], 0.0)
    h = jnp.maximum(_mm_t(h, bw1[...]) + bb1[...], 0.0)
    h = jnp.maximum(_mm_t(h, bw2[...]) + bb2[...], 0.0)
    h_bf = h.astype(_BF)
    h_ref[...] = h
    mask = mask_ref[...]
    srep = srep_ref[...]

    def grp(g, _):
        hg = h_ref[pl.ds(g * _GRP, _GRP), :]
        eg = jnp.reshape(
            emb_ref[:, pl.ds(g * _GRP, _GRP), :], (_NS * _GRP, _D)
        )
        rg = jnp.concatenate([hg, eg], axis=0).astype(_BF)
        mg = _mm_t(rg, rg).astype(_BF) * mask
        gc = lax.dot_general(
            mg, srep, (((1,), (0,)), ((), ())),
            preferred_element_type=jnp.float32,
        )
        g_ref[0, pl.ds(g * _GRP, _GRP), :] = gc[: _GRP, :]
        g_ref[1:, pl.ds(g * _GRP, _GRP), :] = jnp.reshape(
            gc[_GRP:, :], (_NS, _GRP, 32)
        )
        return 0

    lax.fori_loop(0, ngrp, grp, 0, unroll=32)

    z = _mm_t(h_bf, w0h[...])
    for a in range(_NQ):
        pa = None
        for q in range(4):
            f = 4 * a + q
            if f >= _NF:
                break
            t = lax.dot_general(
                g_ref[f].astype(_BF), eq_ref[q],
                (((1,), (0,)), ((), ())),
                preferred_element_type=jnp.float32,
            )
            pa = t if pa is None else pa + t
        z = z + _mm_t(pa.astype(_BF), wsym[pl.ds(a * 1024, 1024), :])
    z = jnp.maximum(z + tb0[...], 0.0)
    z = jnp.maximum(_mm_t(z.astype(_BF), tw1[...]) + tb1[...], 0.0)
    z = jnp.maximum(_mm_t(z.astype(_BF), tw2[...]) + tb2[...], 0.0)
    z = jnp.maximum(_mm_t(z.astype(_BF), tw3[...]) + tb3[...], 0.0)
    out_ref[...] = _mm_t(z.astype(_BF), tw4[...]) + tb4[0, 0]


def _tc_dense(xd, emb3, bw0, bb0, bw1, bb1, bw2, bb2, mask, srep, eq,
              w0h, wsym, tb0, tw1, tb1, tw2, tb2, tw3, tb3, tw4, tb4):
    b = xd.shape[0]
    bb = 512
    nblk = b // bb

    def _full(a):
        return pl.BlockSpec(a.shape, lambda i, _n=None, _nd=a.ndim: (0,) * _nd)

    ws = (bw0, bb0, bw1, bb1, bw2, bb2, mask, srep, eq,
          w0h, wsym, tb0, tw1, tb1, tw2, tb2, tw3, tb3, tw4, tb4)
    return pl.pallas_call(
        _dense_body,
        grid=(nblk,),
        in_specs=[
            pl.BlockSpec((bb, _ND), lambda i: (i, 0)),
            pl.BlockSpec((_NS, bb, _D), lambda i: (0, i, 0)),
        ] + [_full(w) for w in ws],
        out_specs=pl.BlockSpec((bb, 8), lambda i: (i, 0)),
        out_shape=jax.ShapeDtypeStruct((b, 8), jnp.float32),
        scratch_shapes=[
            pltpu.VMEM((bb, _D), jnp.float32),
            pltpu.VMEM((_NF, bb, 32), jnp.float32),
        ],
    )(xd, emb3, *ws)


def kernel(x, table, bot_w0, bot_b0, bot_w1, bot_b1, bot_w2, bot_b2,
           top_w0, top_b0, top_w1, top_b1, top_w2, top_b2,
           top_w3, top_b3, top_w4, top_b4):
    b = x.shape[0]
    xd = x[:, :_ND]
    idxt = x[:, _ND:].astype(jnp.int32).T % _V
    bf = lambda v: v.astype(_BF)
    w0h = bf(top_w0[:, :_D])
    wsym = jnp.einsum(
        "op,pam->aom", top_w0[:, _D:], jnp.asarray(_SELP),
        preferred_element_type=jnp.float32,
    )
    wsym = bf(jnp.reshape(wsym, (_NQ * 1024, _D)))
    tw4p = jnp.zeros((8, top_w4.shape[1]), jnp.float32).at[0].set(top_w4[0])
    r2 = lambda v: jnp.reshape(v, (1, -1))
    wargs = (
        bot_w0, r2(bot_b0), bot_w1, r2(bot_b1), bot_w2, r2(bot_b2),
        bf(jnp.asarray(_MASK)), bf(jnp.asarray(_SREP)), bf(jnp.asarray(_EQ)),
        w0h, wsym, r2(top_b0), bf(top_w1), r2(top_b1), bf(top_w2), r2(top_b2),
        bf(top_w3), r2(top_b3), bf(tw4p), r2(top_b4),
    )
    hb = b // 2
    outs = []
    for i in range(2):
        idx_i = jnp.reshape(idxt[:, i * hb : (i + 1) * hb], (-1,))
        emb_i = _sc_gather(table, idx_i)
        emb3_i = jnp.reshape(emb_i, (_NS, hb, _D))
        outs.append(_tc_dense(xd[i * hb : (i + 1) * hb], emb3_i, *wargs))
    return jnp.concatenate(outs, axis=0)[:, :1]
